# Initial kernel scaffold; baseline (speedup 1.0000x reference)
import jax
import jax.numpy as jnp
from jax.experimental import pallas as pl


def kernel(edge_index, node_attr, edge_attr, batch, W_mpl, b_mpl, W1, b1, W2, b2, W3, b3, W4, b4):
    raise NotImplementedError("write your pallas kernel here")


# speedup vs baseline: 3.3385x; 3.3385x over previous
import functools

import jax
import jax.numpy as jnp
from jax import lax
from jax.experimental import pallas as pl
from jax.experimental.pallas import tpu as pltpu
from jax.experimental.pallas import tpu_sc as plsc

N = 10000
E = 320000
D = 128
DE = 16
G = 64
P = 32
CH = 128
NCHUNK = E // CH
NC = 2
NS = 16
NW = NC * NS
NROW = 10240
RPS = NROW // NS
MAXCH_W = (NCHUNK + NW - 1) // NW

_ILV = [i // 2 if i % 2 == 0 else 16 + i // 2 for i in range(P)]


def _nproj_body(na_ref, w_ref, o_ref):
    o_ref[...] = jnp.dot(na_ref[...], w_ref[...],
                         preferred_element_type=jnp.float32
                         ).astype(jnp.bfloat16)


def _eproj_body(ea_ref, w_ref, b_ref, o_ref):
    o_ref[...] = jnp.dot(ea_ref[...], w_ref[...],
                         preferred_element_type=jnp.float32) + b_ref[...]


def _tc_prologue(node_attr, edge_attr, Wn, We, bm):
    nproj = pl.pallas_call(
        _nproj_body,
        out_shape=jax.ShapeDtypeStruct((N, P), jnp.bfloat16),
    )(node_attr, Wn)

    EB = E // 32
    eproj = pl.pallas_call(
        _eproj_body,
        grid=(32,),
        in_specs=[
            pl.BlockSpec((EB, DE), lambda i: (i, 0)),
            pl.BlockSpec((DE, P), lambda i: (0, 0)),
            pl.BlockSpec((1, P), lambda i: (0, 0)),
        ],
        out_specs=pl.BlockSpec((EB, P), lambda i: (i, 0)),
        out_shape=jax.ShapeDtypeStruct((E, P), jnp.float32),
    )(edge_attr, We, bm)
    return nproj, eproj


def _sc_edge_pass(src_idx, dst_idx, nproj, eproj):
    mesh = plsc.VectorSubcoreMesh(
        core_axis_name="c", subcore_axis_name="s",
        num_cores=NC, num_subcores=NS)

    @functools.partial(
        pl.kernel,
        out_type=jax.ShapeDtypeStruct((NC, NROW, P), jnp.float32),
        mesh=mesh,
        compiler_params=pltpu.CompilerParams(
            needs_layout_passes=False, use_tc_tiling_on_sc=False),
        scratch_types=[
            pltpu.VMEM((CH,), jnp.int32),
            pltpu.VMEM((CH,), jnp.int32),
            pltpu.VMEM((CH, P // 2), jnp.int32),
            pltpu.VMEM((CH, P), jnp.float32),
            pltpu.VMEM((RPS, P), jnp.float32),
            pltpu.VMEM_SHARED((NROW, P), jnp.float32),
            pltpu.SemaphoreType.DMA,
        ],
    )
    def k(src_hbm, dst_hbm, nproj_hbm, eproj_hbm, out_hbm,
          src_v, dst_v, gath_v, ep_v, buf_v, acc_sh, sem):
        c = lax.axis_index("c")
        s = lax.axis_index("s")
        w = s * NC + c

        z = jnp.zeros((16,), jnp.float32)

        def zrow(r, carry):
            buf_v[r, pl.ds(0, 16)] = z
            buf_v[r, pl.ds(16, 16)] = z
            return carry

        lax.fori_loop(0, RPS, zrow, 0)
        r0 = s * RPS
        pltpu.sync_copy(buf_v, acc_sh.at[pl.ds(r0, RPS)])
        plsc.subcore_barrier()

        def chunk(j, carry):
            cidx = j * NW + w

            @pl.when(cidx < NCHUNK)
            def _():
                base = cidx * CH
                pltpu.sync_copy(src_hbm.at[pl.ds(base, CH)], src_v)
                pltpu.sync_copy(dst_hbm.at[pl.ds(base, CH)], dst_v)
                pltpu.sync_copy(eproj_hbm.at[pl.ds(base, CH)], ep_v)
                pltpu.async_copy(nproj_hbm.at[src_v], gath_v, sem).wait()

                def ebody(e, cc):
                    g = plsc.bitcast(gath_v[e, :], jnp.bfloat16)
                    a, b = plsc.unpack(g, format=plsc.PackFormat.INTERLEAVED,
                                       preferred_element_type=jnp.float32)
                    p0 = ep_v[e, pl.ds(0, 16)]
                    p1 = ep_v[e, pl.ds(16, 16)]
                    ep_v[e, pl.ds(0, 16)] = jnp.maximum(a + p0, 0.0)
                    ep_v[e, pl.ds(16, 16)] = jnp.maximum(b + p1, 0.0)
                    return cc

                lax.fori_loop(0, CH, ebody, 0)
                pltpu.sync_copy(ep_v, acc_sh.at[dst_v], add=True)

            return carry

        lax.fori_loop(0, MAXCH_W, chunk, 0)

        plsc.subcore_barrier()
        pltpu.sync_copy(acc_sh.at[pl.ds(r0, RPS)], buf_v)
        pltpu.sync_copy(buf_v, out_hbm.at[c, pl.ds(r0, RPS)])

    return k(src_idx, dst_idx, nproj, eproj)


def _post_body(acc_ref, batch_ref, w1_ref, b1_ref, w2_ref, b2_ref,
               w3_ref, b3_ref, w4_ref, b4_ref, o_ref):
    x = acc_ref[0] + acc_ref[1]
    h1 = jnp.maximum(
        jnp.dot(x, w1_ref[...], preferred_element_type=jnp.float32)
        + b1_ref[...], 0.0)
    h2 = jnp.maximum(
        jnp.dot(h1, w2_ref[...], preferred_element_type=jnp.float32)
        + b2_ref[...], 0.0)
    oh = (batch_ref[...] == lax.broadcasted_iota(jnp.int32, (NROW, G), 1)
          ).astype(jnp.float32)
    pooled = lax.dot_general(oh, h2, (((0,), (0,)), ((), ())),
                             preferred_element_type=jnp.float32)
    g3 = jnp.maximum(
        jnp.dot(pooled, w3_ref[...], preferred_element_type=jnp.float32)
        + b3_ref[...], 0.0)
    o_ref[...] = jnp.dot(g3, w4_ref[...],
                         preferred_element_type=jnp.float32) + b4_ref[...]


def _tc_epilogue(acc2, batch2d, W1p, b1p, W2p, b2p, W3p, b3p, W4p, b4p):
    return pl.pallas_call(
        _post_body,
        out_shape=jax.ShapeDtypeStruct((G, 8), jnp.float32),
    )(acc2, batch2d, W1p, b1p, W2p, b2p, W3p, b3p, W4p, b4p)


def kernel(edge_index, node_attr, edge_attr, batch,
           W_mpl, b_mpl, W1, b1, W2, b2, W3, b3, W4, b4):
    Wn = jnp.pad(W_mpl[:D], ((0, 0), (0, P - 20)))[:, jnp.array(_ILV)]
    We = jnp.pad(W_mpl[D:], ((0, 0), (0, P - 20)))
    bm = jnp.pad(b_mpl, (0, P - 20)).reshape(1, P)
    W1p = jnp.pad(W1, ((0, 12), (0, 12)))
    b1p = jnp.pad(b1, (0, 12)).reshape(1, 32)
    W2p = jnp.pad(W2, ((0, 12), (0, 6)))
    b2p = jnp.pad(b2, (0, 6)).reshape(1, 16)
    W3p = jnp.pad(W3, ((0, 6), (0, 6)))
    b3p = jnp.pad(b3, (0, 6)).reshape(1, 16)
    W4p = jnp.pad(W4, ((0, 6), (0, 7)))
    b4p = jnp.pad(b4, (0, 7)).reshape(1, 8)

    nproj, eproj = _tc_prologue(node_attr, edge_attr, Wn, We, bm)
    nproj = lax.bitcast_convert_type(nproj.reshape(N, P // 2, 2), jnp.int32)
    nproj = jnp.pad(nproj, ((0, NROW - N), (0, 0)))
    acc2 = _sc_edge_pass(edge_index[0], edge_index[1], nproj, eproj)
    batch_pad = jnp.pad(batch, (0, NROW - N), constant_values=G).reshape(NROW, 1)
    out = _tc_epilogue(acc2, batch_pad,
                       W1p, b1p, W2p, b2p, W3p, b3p, W4p, b4p)
    return out[:, :1]
